# Initial kernel scaffold; baseline (speedup 1.0000x reference)
import jax
import jax.numpy as jnp
from jax import lax
from jax.experimental import pallas as pl
from jax.experimental.pallas import tpu as pltpu
from jax.experimental.pallas import tpu_sc as plsc

NU = 25000
NI = 25000
N = NU + NI
D = 64
E = 800000
N_LAYERS = 3

NC = 2
NS = 16
BLK = 128
CHUNK = 16
BLOCKS_PER_TILE = 400
E_PAD = NS * BLOCKS_PER_TILE * BLK
CHUNKS_PER_TILE = BLOCKS_PER_TILE // CHUNK

HALF = N // NC
ACC_ROWS = 25088
ROWS_PER_TILE = ACC_ROWS // NS
LAST_DRAIN = HALF - (NS - 1) * ROWS_PER_TILE


def _layer_body(table, src2d, dst2d, w2d, zeros, out,
                acc, src_c, ldst_c, w_c, rows, sem):
    c = lax.axis_index("c")
    s = lax.axis_index("s")
    base = c * HALF

    zstart = s * ROWS_PER_TILE
    pltpu.sync_copy(zeros.at[pl.ds(zstart, ROWS_PER_TILE)],
                    acc.at[pl.ds(zstart, ROWS_PER_TILE)])
    plsc.subcore_barrier()

    def chunk_body(k, _):
        row0 = s * BLOCKS_PER_TILE + k * CHUNK
        pltpu.sync_copy(src2d.at[pl.ds(row0, CHUNK)], src_c)
        pltpu.sync_copy(dst2d.at[pl.ds(row0, CHUNK)], ldst_c)
        pltpu.sync_copy(w2d.at[pl.ds(row0, CHUNK)], w_c)

        def ldst_body(i, _):
            r = i // (BLK // 16)
            col = (i % (BLK // 16)) * 16
            d = ldst_c[r, pl.ds(col, 16)] - base
            ok = (d >= 0) & (d < HALF)
            ldst_c[r, pl.ds(col, 16)] = jnp.where(ok, d, HALF)
            return 0
        lax.fori_loop(0, CHUNK * BLK // 16, ldst_body, 0)

        def blk_body(b, _):
            pltpu.async_copy(table.at[src_c.at[b]], rows, sem).wait()

            def scale_body(e, _):
                wv = w_c[b, e]
                for j in range(D // 16):
                    rows[e, pl.ds(j * 16, 16)] = rows[e, pl.ds(j * 16, 16)] * wv
                return 0
            lax.fori_loop(0, BLK, scale_body, 0)

            pltpu.sync_copy(rows, acc.at[ldst_c.at[b]], add=True)
            return 0
        lax.fori_loop(0, CHUNK, blk_body, 0)
        return 0
    lax.fori_loop(0, CHUNKS_PER_TILE, chunk_body, 0)

    plsc.subcore_barrier()

    dstart = s * ROWS_PER_TILE

    @pl.when(s < NS - 1)
    def _():
        pltpu.sync_copy(acc.at[pl.ds(dstart, ROWS_PER_TILE)],
                        out.at[pl.ds(base + dstart, ROWS_PER_TILE)])

    @pl.when(s == NS - 1)
    def _():
        pltpu.sync_copy(acc.at[pl.ds(dstart, LAST_DRAIN)],
                        out.at[pl.ds(base + dstart, LAST_DRAIN)])


_layer = pl.kernel(
    _layer_body,
    out_type=jax.ShapeDtypeStruct((N, D), jnp.float32),
    mesh=plsc.VectorSubcoreMesh(core_axis_name="c", subcore_axis_name="s"),
    scratch_types=[
        pltpu.VMEM_SHARED((ACC_ROWS, D), jnp.float32),
        pltpu.VMEM((CHUNK, BLK), jnp.int32),
        pltpu.VMEM((CHUNK, BLK), jnp.int32),
        pltpu.VMEM((CHUNK, BLK), jnp.float32),
        pltpu.VMEM((BLK, D), jnp.float32),
        pltpu.SemaphoreType.DMA,
    ],
)


def _mean_body(a, b, c, d, o):
    o[...] = (a[...] + b[...] + c[...] + d[...]) * 0.25


_MEAN_ROWS = 2000


def _mean4(e0, e1, e2, e3):
    spec = pl.BlockSpec((_MEAN_ROWS, D), lambda i: (i, 0))
    return pl.pallas_call(
        _mean_body,
        grid=(N // _MEAN_ROWS,),
        in_specs=[spec] * 4,
        out_specs=spec,
        out_shape=jax.ShapeDtypeStruct((N, D), jnp.float32),
    )(e0, e1, e2, e3)


def kernel(user_emb, item_emb, edge_index, edge_weight):
    emb0 = jnp.concatenate([user_emb, item_emb], axis=0)
    pad = E_PAD - E
    src = jnp.concatenate(
        [edge_index[0], jnp.zeros((pad,), jnp.int32)]).reshape(E_PAD // BLK, BLK)
    dst = jnp.concatenate(
        [edge_index[1], jnp.zeros((pad,), jnp.int32)]).reshape(E_PAD // BLK, BLK)
    w = jnp.concatenate(
        [edge_weight, jnp.zeros((pad,), jnp.float32)]).reshape(E_PAD // BLK, BLK)
    zeros = jnp.zeros((ACC_ROWS, D), jnp.float32)

    embs = [emb0]
    for _ in range(N_LAYERS):
        embs.append(_layer(embs[-1], src, dst, w, zeros))
    light_out = _mean4(*embs)
    return light_out[:NU], light_out[NU:]


# speedup vs baseline: 2.0446x; 2.0446x over previous
import jax
import jax.numpy as jnp
from jax import lax
from jax.experimental import pallas as pl
from jax.experimental.pallas import tpu as pltpu
from jax.experimental.pallas import tpu_sc as plsc

NU = 25000
NI = 25000
N = NU + NI
D = 64
E = 800000
N_LAYERS = 3

NC = 2
NS = 16
BLK = 128
CHUNK = 16
BLOCKS_PER_TILE = 400
E_PAD = NS * BLOCKS_PER_TILE * BLK
CHUNKS_PER_TILE = BLOCKS_PER_TILE // CHUNK

HALF = N // NC
ACC_ROWS = 25088
ROWS_PER_TILE = ACC_ROWS // NS
LAST_DRAIN = HALF - (NS - 1) * ROWS_PER_TILE


def _layer_body(table, src2d, dst2d, w2d, zeros, out,
                acc, src_c, ldst_c, w_c, rows, sem):
    c = lax.axis_index("c")
    s = lax.axis_index("s")
    base = c * HALF

    zstart = s * ROWS_PER_TILE
    pltpu.sync_copy(zeros.at[pl.ds(zstart, ROWS_PER_TILE)],
                    acc.at[pl.ds(zstart, ROWS_PER_TILE)])
    plsc.subcore_barrier()

    def chunk_body(k, _):
        row0 = s * BLOCKS_PER_TILE + k * CHUNK
        pltpu.sync_copy(src2d.at[pl.ds(row0, CHUNK)], src_c)
        pltpu.sync_copy(dst2d.at[pl.ds(row0, CHUNK)], ldst_c)
        pltpu.sync_copy(w2d.at[pl.ds(row0, CHUNK)], w_c)

        def ldst_body(i, _):
            r = i // (BLK // 16)
            col = (i % (BLK // 16)) * 16
            d = ldst_c[r, pl.ds(col, 16)] - base
            ok = (d >= 0) & (d < HALF)
            ldst_c[r, pl.ds(col, 16)] = jnp.where(ok, d, HALF)
            return 0
        lax.fori_loop(0, CHUNK * BLK // 16, ldst_body, 0)

        def blk_body(b, _):
            pltpu.async_copy(table.at[src_c.at[b]], rows, sem).wait()

            def scale_body(g, _):
                wvec = w_c[b, pl.ds(g * 16, 16)]
                for e16 in range(16):
                    e = g * 16 + e16
                    wv = wvec.at[jnp.full((16,), e16, jnp.int32)].get(
                        mode="promise_in_bounds")
                    for j in range(D // 16):
                        rows[e, pl.ds(j * 16, 16)] = (
                            rows[e, pl.ds(j * 16, 16)] * wv)
                return 0
            lax.fori_loop(0, BLK // 16, scale_body, 0)

            pltpu.sync_copy(rows, acc.at[ldst_c.at[b]], add=True)
            return 0
        lax.fori_loop(0, CHUNK, blk_body, 0)
        return 0
    lax.fori_loop(0, CHUNKS_PER_TILE, chunk_body, 0)

    plsc.subcore_barrier()

    dstart = s * ROWS_PER_TILE

    @pl.when(s < NS - 1)
    def _():
        pltpu.sync_copy(acc.at[pl.ds(dstart, ROWS_PER_TILE)],
                        out.at[pl.ds(base + dstart, ROWS_PER_TILE)])

    @pl.when(s == NS - 1)
    def _():
        pltpu.sync_copy(acc.at[pl.ds(dstart, LAST_DRAIN)],
                        out.at[pl.ds(base + dstart, LAST_DRAIN)])


_layer = pl.kernel(
    _layer_body,
    out_type=jax.ShapeDtypeStruct((N, D), jnp.float32),
    mesh=plsc.VectorSubcoreMesh(core_axis_name="c", subcore_axis_name="s"),
    compiler_params=pltpu.CompilerParams(use_tc_tiling_on_sc=False),
    scratch_types=[
        pltpu.VMEM_SHARED((ACC_ROWS, D), jnp.float32),
        pltpu.VMEM((CHUNK, BLK), jnp.int32),
        pltpu.VMEM((CHUNK, BLK), jnp.int32),
        pltpu.VMEM((CHUNK, BLK), jnp.float32),
        pltpu.VMEM((BLK, D), jnp.float32),
        pltpu.SemaphoreType.DMA,
    ],
)


def _mean_body(a, b, c, d, o):
    o[...] = (a[...] + b[...] + c[...] + d[...]) * 0.25


_MEAN_ROWS = 2000


def _mean4(e0, e1, e2, e3):
    spec = pl.BlockSpec((_MEAN_ROWS, D), lambda i: (i, 0))
    return pl.pallas_call(
        _mean_body,
        grid=(N // _MEAN_ROWS,),
        in_specs=[spec] * 4,
        out_specs=spec,
        out_shape=jax.ShapeDtypeStruct((N, D), jnp.float32),
    )(e0, e1, e2, e3)


def kernel(user_emb, item_emb, edge_index, edge_weight):
    emb0 = jnp.concatenate([user_emb, item_emb], axis=0)
    pad = E_PAD - E
    src = jnp.concatenate(
        [edge_index[0], jnp.zeros((pad,), jnp.int32)]).reshape(E_PAD // BLK, BLK)
    dst = jnp.concatenate(
        [edge_index[1], jnp.zeros((pad,), jnp.int32)]).reshape(E_PAD // BLK, BLK)
    w = jnp.concatenate(
        [edge_weight, jnp.zeros((pad,), jnp.float32)]).reshape(E_PAD // BLK, BLK)
    zeros = jnp.zeros((ACC_ROWS, D), jnp.float32)

    embs = [emb0]
    for _ in range(N_LAYERS):
        embs.append(_layer(embs[-1], src, dst, w, zeros))
    light_out = _mean4(*embs)
    return light_out[:NU], light_out[NU:]


# speedup vs baseline: 2.8431x; 1.3905x over previous
import jax
import jax.numpy as jnp
from jax import lax
from jax.experimental import pallas as pl
from jax.experimental.pallas import tpu as pltpu
from jax.experimental.pallas import tpu_sc as plsc

NU = 25000
NI = 25000
N = NU + NI
D = 64
E = 800000
N_LAYERS = 3

NC = 2
NS = 16
NW = NC * NS
PASSES = D // NW

BLK = 128
CH_ROWS = 16
E_PAD = 819200
NROWS = E_PAD // BLK
NCHUNKS = NROWS // CH_ROWS


def _layer_body(tableT, sd2d, w2d, outT,
                colb, acc, sd0, sd1, w0, w1, sem0, sem1):
    c = lax.axis_index("c")
    s = lax.axis_index("s")
    wid = s * NC + c
    sdbufs = (sd0, sd1)
    wbufs = (w0, w1)
    sems = (sem0, sem1)
    zero16 = jnp.zeros((16,), jnp.float32)
    sh16 = jnp.full((16,), 16, jnp.int32)
    m16 = jnp.full((16,), 0xFFFF, jnp.int32)

    for p in range(PASSES):
        col = wid + NW * p
        pltpu.sync_copy(tableT.at[col], colb)

        @pl.loop(0, N // 16)
        def _(i):
            acc[pl.ds(i * 16, 16)] = zero16

        pltpu.async_copy(sd2d.at[pl.ds(0, CH_ROWS)], sd0, sem0)
        pltpu.async_copy(w2d.at[pl.ds(0, CH_ROWS)], w0, sem0)

        @pl.loop(0, NCHUNKS, step=2)
        def _(k2):
            for par in range(2):
                k = k2 + par
                sdb, wb, sm = sdbufs[par], wbufs[par], sems[par]
                nsdb, nwb, nsm = sdbufs[1 - par], wbufs[1 - par], sems[1 - par]

                @pl.when(k + 1 < NCHUNKS)
                def _():
                    row0 = (k + 1) * CH_ROWS
                    pltpu.async_copy(sd2d.at[pl.ds(row0, CH_ROWS)], nsdb, nsm)
                    pltpu.async_copy(w2d.at[pl.ds(row0, CH_ROWS)], nwb, nsm)

                pltpu.make_async_copy(
                    sd2d.at[pl.ds(0, CH_ROWS)], sdb, sm).wait()
                pltpu.make_async_copy(
                    w2d.at[pl.ds(0, CH_ROWS)], wb, sm).wait()

                @pl.loop(0, CH_ROWS)
                def _(r):
                    for v in range(BLK // 16):
                        sdv = sdb[r, pl.ds(v * 16, 16)]
                        wv = wb[r, pl.ds(v * 16, 16)]
                        srcv = sdv & m16
                        dstv = lax.shift_right_logical(sdv, sh16)
                        g = plsc.load_gather(colb, [srcv])
                        plsc.addupdate_scatter(acc, [dstv], g * wv)

        pltpu.sync_copy(acc, outT.at[col])


_layer = pl.kernel(
    _layer_body,
    out_type=jax.ShapeDtypeStruct((D, N), jnp.float32),
    mesh=plsc.VectorSubcoreMesh(core_axis_name="c", subcore_axis_name="s"),
    compiler_params=pltpu.CompilerParams(use_tc_tiling_on_sc=False,
                                         needs_layout_passes=False),
    scratch_types=[
        pltpu.VMEM((N,), jnp.float32),
        pltpu.VMEM((N,), jnp.float32),
        pltpu.VMEM((CH_ROWS, BLK), jnp.int32),
        pltpu.VMEM((CH_ROWS, BLK), jnp.int32),
        pltpu.VMEM((CH_ROWS, BLK), jnp.float32),
        pltpu.VMEM((CH_ROWS, BLK), jnp.float32),
        pltpu.SemaphoreType.DMA,
        pltpu.SemaphoreType.DMA,
    ],
)


def _mean_body(a, b, c, d, o):
    o[...] = (a[...] + b[...] + c[...] + d[...]) * 0.25


def _mean4(e0, e1, e2, e3):
    spec = pl.BlockSpec((D // 4, N), lambda i: (i, 0))
    return pl.pallas_call(
        _mean_body,
        grid=(4,),
        in_specs=[spec] * 4,
        out_specs=spec,
        out_shape=jax.ShapeDtypeStruct((D, N), jnp.float32),
    )(e0, e1, e2, e3)


def kernel(user_emb, item_emb, edge_index, edge_weight):
    embT0 = jnp.concatenate([user_emb, item_emb], axis=0).T

    pad = E_PAD - E
    src = jnp.concatenate([edge_index[0], jnp.zeros((pad,), jnp.int32)])
    dst = jnp.concatenate([edge_index[1], jnp.zeros((pad,), jnp.int32)])
    sd = (src | (dst << 16)).reshape(NROWS, BLK)
    w = jnp.concatenate(
        [edge_weight, jnp.zeros((pad,), jnp.float32)]).reshape(NROWS, BLK)

    embs = [embT0]
    for _ in range(N_LAYERS):
        embs.append(_layer(embs[-1], sd, w))
    light_out = _mean4(*embs).T
    return light_out[:NU], light_out[NU:]


# speedup vs baseline: 6.1583x; 2.1661x over previous
import jax
import jax.numpy as jnp
from jax import lax
from jax.experimental import pallas as pl
from jax.experimental.pallas import tpu as pltpu
from jax.experimental.pallas import tpu_sc as plsc

NU = 25000
NI = 25000
N = NU + NI
D = 64
E = 800000
N_LAYERS = 3

NC = 2
NS = 16
NW = NC * NS
PASSES = D // NW

BLK = 128
CH_ROWS = 16
E_PAD = 819200
NROWS = E_PAD // BLK
NCHUNKS = NROWS // CH_ROWS


def _layer_body(tableT, sd2d, w2d, outT,
                colb, acc, sd0, sd1, w0, w1, sem0, sem1):
    c = lax.axis_index("c")
    s = lax.axis_index("s")
    wid = s * NC + c
    sdbufs = (sd0, sd1)
    wbufs = (w0, w1)
    sems = (sem0, sem1)
    zero16 = jnp.zeros((16,), jnp.float32)
    sh16 = jnp.full((16,), 16, jnp.int32)
    m16 = jnp.full((16,), 0xFFFF, jnp.int32)

    for p in range(PASSES):
        col = wid + NW * p
        pltpu.sync_copy(tableT.at[col], colb)

        @pl.loop(0, N // 16)
        def _(i):
            acc[pl.ds(i * 16, 16)] = zero16

        pltpu.async_copy(sd2d.at[pl.ds(0, CH_ROWS)], sd0, sem0)
        pltpu.async_copy(w2d.at[pl.ds(0, CH_ROWS)], w0, sem0)

        @pl.loop(0, NCHUNKS, step=2)
        def _(k2):
            for par in range(2):
                k = k2 + par
                sdb, wb, sm = sdbufs[par], wbufs[par], sems[par]
                nsdb, nwb, nsm = sdbufs[1 - par], wbufs[1 - par], sems[1 - par]

                @pl.when(k + 1 < NCHUNKS)
                def _():
                    row0 = (k + 1) * CH_ROWS
                    pltpu.async_copy(sd2d.at[pl.ds(row0, CH_ROWS)], nsdb, nsm)
                    pltpu.async_copy(w2d.at[pl.ds(row0, CH_ROWS)], nwb, nsm)

                pltpu.make_async_copy(
                    sd2d.at[pl.ds(0, CH_ROWS)], sdb, sm).wait()
                pltpu.make_async_copy(
                    w2d.at[pl.ds(0, CH_ROWS)], wb, sm).wait()

                @plsc.parallel_loop(0, CH_ROWS)
                def _(r):
                    for v in range(BLK // 16):
                        sdv = sdb[r, pl.ds(v * 16, 16)]
                        wv = wb[r, pl.ds(v * 16, 16)]
                        srcv = sdv & m16
                        dstv = lax.shift_right_logical(sdv, sh16)
                        g = plsc.load_gather(colb, [srcv])
                        plsc.addupdate_scatter(acc, [dstv], g * wv)

        pltpu.sync_copy(acc, outT.at[col])


_layer = pl.kernel(
    _layer_body,
    out_type=jax.ShapeDtypeStruct((D, N), jnp.float32),
    mesh=plsc.VectorSubcoreMesh(core_axis_name="c", subcore_axis_name="s"),
    compiler_params=pltpu.CompilerParams(use_tc_tiling_on_sc=False,
                                         needs_layout_passes=False),
    scratch_types=[
        pltpu.VMEM((N,), jnp.float32),
        pltpu.VMEM((N,), jnp.float32),
        pltpu.VMEM((CH_ROWS, BLK), jnp.int32),
        pltpu.VMEM((CH_ROWS, BLK), jnp.int32),
        pltpu.VMEM((CH_ROWS, BLK), jnp.float32),
        pltpu.VMEM((CH_ROWS, BLK), jnp.float32),
        pltpu.SemaphoreType.DMA,
        pltpu.SemaphoreType.DMA,
    ],
)


def _mean_body(a, b, c, d, o):
    o[...] = (a[...] + b[...] + c[...] + d[...]) * 0.25


def _mean4(e0, e1, e2, e3):
    spec = pl.BlockSpec((D // 4, N), lambda i: (i, 0))
    return pl.pallas_call(
        _mean_body,
        grid=(4,),
        in_specs=[spec] * 4,
        out_specs=spec,
        out_shape=jax.ShapeDtypeStruct((D, N), jnp.float32),
    )(e0, e1, e2, e3)


def kernel(user_emb, item_emb, edge_index, edge_weight):
    embT0 = jnp.concatenate([user_emb, item_emb], axis=0).T

    pad = E_PAD - E
    src = jnp.concatenate([edge_index[0], jnp.zeros((pad,), jnp.int32)])
    dst = jnp.concatenate([edge_index[1], jnp.zeros((pad,), jnp.int32)])
    sd = (src | (dst << 16)).reshape(NROWS, BLK)
    w = jnp.concatenate(
        [edge_weight, jnp.zeros((pad,), jnp.float32)]).reshape(NROWS, BLK)

    embs = [embT0]
    for _ in range(N_LAYERS):
        embs.append(_layer(embs[-1], sd, w))
    light_out = _mean4(*embs).T
    return light_out[:NU], light_out[NU:]


# speedup vs baseline: 7.6955x; 1.2496x over previous
import jax
import jax.numpy as jnp
from jax import lax
from jax.experimental import pallas as pl
from jax.experimental.pallas import tpu as pltpu
from jax.experimental.pallas import tpu_sc as plsc

NU = 25000
NI = 25000
N = NU + NI
D = 64
E = 800000
N_LAYERS = 3

NC = 2
NS = 16
NW = NC * NS
PASSES = D // NW

BLK = 128
CH_ROWS = 32
E_PAD = 819200
NROWS = E_PAD // BLK
NCHUNKS = NROWS // CH_ROWS


def _layer_body(tableT, sd2d, w2d, outT,
                colb, acc, sd0, sd1, w0, w1, sem0, sem1):
    c = lax.axis_index("c")
    s = lax.axis_index("s")
    wid = s * NC + c
    sdbufs = (sd0, sd1)
    wbufs = (w0, w1)
    sems = (sem0, sem1)
    zero16 = jnp.zeros((16,), jnp.float32)
    sh16 = jnp.full((16,), 16, jnp.int32)
    m16 = jnp.full((16,), 0xFFFF, jnp.int32)

    for p in range(PASSES):
        col = wid + NW * p
        pltpu.sync_copy(tableT.at[col], colb)

        @pl.loop(0, N // 16)
        def _(i):
            acc[pl.ds(i * 16, 16)] = zero16

        pltpu.async_copy(sd2d.at[pl.ds(0, CH_ROWS)], sd0, sem0)
        pltpu.async_copy(w2d.at[pl.ds(0, CH_ROWS)], w0, sem0)

        @pl.loop(0, NCHUNKS, step=2)
        def _(k2):
            for par in range(2):
                k = k2 + par
                sdb, wb, sm = sdbufs[par], wbufs[par], sems[par]
                nsdb, nwb, nsm = sdbufs[1 - par], wbufs[1 - par], sems[1 - par]

                @pl.when(k + 1 < NCHUNKS)
                def _():
                    row0 = (k + 1) * CH_ROWS
                    pltpu.async_copy(sd2d.at[pl.ds(row0, CH_ROWS)], nsdb, nsm)
                    pltpu.async_copy(w2d.at[pl.ds(row0, CH_ROWS)], nwb, nsm)

                pltpu.make_async_copy(
                    sd2d.at[pl.ds(0, CH_ROWS)], sdb, sm).wait()
                pltpu.make_async_copy(
                    w2d.at[pl.ds(0, CH_ROWS)], wb, sm).wait()

                @plsc.parallel_loop(0, CH_ROWS, unroll=2)
                def _(r):
                    for v in range(BLK // 16):
                        sdv = sdb[r, pl.ds(v * 16, 16)]
                        wv = wb[r, pl.ds(v * 16, 16)]
                        srcv = sdv & m16
                        dstv = lax.shift_right_logical(sdv, sh16)
                        g = plsc.load_gather(colb, [srcv])
                        plsc.addupdate_scatter(acc, [dstv], g * wv)

        pltpu.sync_copy(acc, outT.at[col])


_layer = pl.kernel(
    _layer_body,
    out_type=jax.ShapeDtypeStruct((D, N), jnp.float32),
    mesh=plsc.VectorSubcoreMesh(core_axis_name="c", subcore_axis_name="s"),
    compiler_params=pltpu.CompilerParams(use_tc_tiling_on_sc=False,
                                         needs_layout_passes=False),
    scratch_types=[
        pltpu.VMEM((N,), jnp.float32),
        pltpu.VMEM((N,), jnp.float32),
        pltpu.VMEM((CH_ROWS, BLK), jnp.int32),
        pltpu.VMEM((CH_ROWS, BLK), jnp.int32),
        pltpu.VMEM((CH_ROWS, BLK), jnp.float32),
        pltpu.VMEM((CH_ROWS, BLK), jnp.float32),
        pltpu.SemaphoreType.DMA,
        pltpu.SemaphoreType.DMA,
    ],
)


def _mean_body(a, b, c, d, o):
    o[...] = (a[...] + b[...] + c[...] + d[...]) * 0.25


def _mean4(e0, e1, e2, e3):
    spec = pl.BlockSpec((D // 4, N), lambda i: (i, 0))
    return pl.pallas_call(
        _mean_body,
        grid=(4,),
        in_specs=[spec] * 4,
        out_specs=spec,
        out_shape=jax.ShapeDtypeStruct((D, N), jnp.float32),
    )(e0, e1, e2, e3)


def kernel(user_emb, item_emb, edge_index, edge_weight):
    embT0 = jnp.concatenate([user_emb, item_emb], axis=0).T

    pad = E_PAD - E
    src = jnp.concatenate([edge_index[0], jnp.zeros((pad,), jnp.int32)])
    dst = jnp.concatenate([edge_index[1], jnp.zeros((pad,), jnp.int32)])
    sd = (src | (dst << 16)).reshape(NROWS, BLK)
    w = jnp.concatenate(
        [edge_weight, jnp.zeros((pad,), jnp.float32)]).reshape(NROWS, BLK)

    embs = [embT0]
    for _ in range(N_LAYERS):
        embs.append(_layer(embs[-1], sd, w))
    light_out = _mean4(*embs).T
    return light_out[:NU], light_out[NU:]
